# baseline (speedup 1.0000x reference)
import functools

import jax
import jax.numpy as jnp
from jax import lax
from jax.experimental import pallas as pl
from jax.experimental.pallas import tpu as pltpu
from jax.experimental.pallas import tpu_sc as plsc

EMBEDDING_DIM = 64
R = 128
G = 2
NBANKS = 4
L = 16
W = 256


@functools.lru_cache(maxsize=None)
def _build_format(V, D, NC, NS):
    NW = NC * NS
    n_full = V // W
    n_tail = V - n_full * W
    T = (n_full + NW - 1) // NW

    mesh = plsc.VectorSubcoreMesh(core_axis_name="c", subcore_axis_name="s")

    T2 = (T + 1) // 2

    @functools.partial(
        pl.kernel,
        mesh=mesh,
        out_type=jax.ShapeDtypeStruct((V * D,), jnp.float32),
        scratch_types=[
            pltpu.VMEM((D, W), jnp.float32),
            pltpu.VMEM((D, W), jnp.float32),
            pltpu.VMEM((W * D,), jnp.float32),
            pltpu.VMEM((W * D,), jnp.float32),
            pltpu.SemaphoreType.DMA,
            pltpu.SemaphoreType.DMA,
            pltpu.SemaphoreType.DMA,
            pltpu.SemaphoreType.DMA,
        ],
        compiler_params=pltpu.CompilerParams(needs_layout_passes=False),
    )
    def format_kernel(wt_hbm, tail_hbm, lin_hbm,
                      in0, in1, out0, out1, is0, is1, os0, os1):
        wid = lax.axis_index("s") * NC + lax.axis_index("c")
        ins, outs = (in0, in1), (out0, out1)
        isems, osems = (is0, is1), (os0, os1)
        l_vec = lax.iota(jnp.int32, L)
        row_vecs = [l_vec + k * L for k in range(D // L)]

        def vstrip(s_idx):
            return jnp.minimum(wid + s_idx * NW, n_full - 1) * W

        def fire_in(s_idx, p):
            pltpu.async_copy(wt_hbm.at[:, pl.ds(vstrip(s_idx), W)],
                             ins[p], isems[p])

        def wait_in(s_idx, p):
            pltpu.make_async_copy(wt_hbm.at[:, pl.ds(vstrip(s_idx), W)],
                                  ins[p], isems[p]).wait()

        def transpose(p):
            def row_body(j_outer, carry2):
                for jj in range(16):
                    j = j_outer * 16 + jj
                    col_vec = (j + l_vec) & (W - 1)
                    sbase = (col_vec << 6) + l_vec
                    for k in range(D // L):
                        vals = plsc.load_gather(ins[p], [row_vecs[k], col_vec])
                        plsc.store_scatter(outs[p], [sbase + k * L], vals)
                return carry2

            lax.fori_loop(0, W // 16, row_body, 0)

        def fire_out(s_idx, p):
            pltpu.async_copy(outs[p], lin_hbm.at[pl.ds(vstrip(s_idx) * D,
                                                       W * D)], osems[p])

        def wait_out(p):
            pltpu.make_async_copy(outs[p], lin_hbm.at[pl.ds(0, W * D)],
                                  osems[p]).wait()

        fire_in(0, 0)
        for p in range(2):
            wait_in(p, p)
            fire_in(p + 1, 1 - p)
            transpose(p)
            fire_out(p, p)

        def body(t2, carry):
            for p in range(2):
                s_idx = 2 * t2 + p
                wait_in(s_idx, p)
                fire_in(s_idx + 1, 1 - p)
                wait_out(p)
                transpose(p)
                fire_out(s_idx, p)
            return carry

        lax.fori_loop(1, T2, body, 0)
        wait_in(2 * T2, 0)
        wait_out(0)
        wait_out(1)

        if n_tail:
            @pl.when(wid == 0)
            def _():
                pltpu.sync_copy(tail_hbm, out0.at[pl.ds(0, n_tail * D)])
                pltpu.sync_copy(out0.at[pl.ds(0, n_tail * D)],
                                lin_hbm.at[pl.ds(n_full * W * D, n_tail * D)])

    return format_kernel


@functools.lru_cache(maxsize=None)
def _build_gather(B, D, NC, NS):
    NW = NC * NS
    b_per_w = B // NW
    S = b_per_w // R
    ngroups = S // G
    T = ngroups // NBANKS

    mesh = plsc.VectorSubcoreMesh(core_axis_name="c", subcore_axis_name="s")

    @functools.partial(
        pl.kernel,
        mesh=mesh,
        out_type=jax.ShapeDtypeStruct((B, D), jnp.float32),
        scratch_types=(
            [pltpu.VMEM((S * R,), jnp.int32)]
            + [pltpu.VMEM((G * R, D), jnp.float32)] * NBANKS
            + [pltpu.SemaphoreType.DMA] * (2 * NBANKS)
        ),
        compiler_params=pltpu.CompilerParams(use_tc_tiling_on_sc=False),
    )
    def gather_kernel(table_hbm, idx_hbm, out_hbm, idx_v, *bufs):
        rows = bufs[:NBANKS]
        gsems = bufs[NBANKS:2 * NBANKS]
        osems = bufs[2 * NBANKS:]
        wid = lax.axis_index("s") * NC + lax.axis_index("c")
        base = wid * b_per_w
        pltpu.sync_copy(idx_hbm.at[pl.ds(base, b_per_w)], idx_v)

        def fire_gathers(g, k):
            for b in range(G):
                s = jnp.minimum(g * G + b, S - 1)
                pltpu.async_copy(table_hbm.at[idx_v.at[pl.ds(s * R, R)]],
                                 rows[k].at[pl.ds(b * R, R)], gsems[k])

        def wait_gathers(g, k):
            for b in range(G):
                s = jnp.minimum(g * G + b, S - 1)
                pltpu.make_async_copy(table_hbm.at[idx_v.at[pl.ds(s * R, R)]],
                                      rows[k].at[pl.ds(b * R, R)], gsems[k]).wait()

        def fire_outs(g, k):
            pltpu.async_copy(rows[k], out_hbm.at[pl.ds(base + g * G * R, G * R)],
                             osems[k])

        def wait_outs(g, k):
            pltpu.make_async_copy(rows[k], out_hbm.at[pl.ds(base + g * G * R, G * R)],
                                  osems[k]).wait()

        for k in range(NBANKS):
            fire_gathers(k, k)

        def body(t, carry):
            g0 = NBANKS * t
            for k in range(NBANKS):
                wait_gathers(g0 + k, k)
                fire_outs(g0 + k, k)
            for k in range(NBANKS):
                wait_outs(g0 + k, k)
                fire_gathers(g0 + NBANKS + k, k)
            return carry

        lax.fori_loop(0, T, body, 0)
        for k in range(NBANKS):
            wait_gathers(NBANKS * T + k, k)

    return gather_kernel


def kernel(token_ids, weight):
    B = token_ids.shape[0] * token_ids.shape[1]
    V, D = weight.shape
    info = plsc.get_sparse_core_info()
    NC, NS = info.num_cores, info.num_subcores
    idx = token_ids.reshape(-1).astype(jnp.int32)
    n_full = V // W
    tail = weight[n_full * W:, :].reshape(-1)
    w_lin = _build_format(V, D, NC, NS)(weight.T, tail)
    out = _build_gather(B, D, NC, NS)(w_lin.reshape(V, D), idx)
    return out.reshape(token_ids.shape[0], token_ids.shape[1], D)


# speedup vs baseline: 1.0139x; 1.0139x over previous
import functools

import jax
import jax.numpy as jnp
from jax import lax
from jax.experimental import pallas as pl
from jax.experimental.pallas import tpu as pltpu
from jax.experimental.pallas import tpu_sc as plsc

EMBEDDING_DIM = 64
R = 128
G = 2
NBANKS = 4
L = 16


@functools.lru_cache(maxsize=None)
def _build_format(V, D, NC, NS):
    NW = NC * NS
    n_full = V // 128
    n_tail = V - n_full * 128
    T = (n_full + NW - 1) // NW

    mesh = plsc.VectorSubcoreMesh(core_axis_name="c", subcore_axis_name="s")

    T2 = (T + 1) // 2

    @functools.partial(
        pl.kernel,
        mesh=mesh,
        out_type=jax.ShapeDtypeStruct((V * D,), jnp.float32),
        scratch_types=[
            pltpu.VMEM((D, 128), jnp.float32),
            pltpu.VMEM((D, 128), jnp.float32),
            pltpu.VMEM((128 * D,), jnp.float32),
            pltpu.VMEM((128 * D,), jnp.float32),
            pltpu.SemaphoreType.DMA,
            pltpu.SemaphoreType.DMA,
            pltpu.SemaphoreType.DMA,
            pltpu.SemaphoreType.DMA,
        ],
        compiler_params=pltpu.CompilerParams(needs_layout_passes=False),
    )
    def format_kernel(wt_hbm, tail_hbm, lin_hbm,
                      in0, in1, out0, out1, is0, is1, os0, os1):
        wid = lax.axis_index("s") * NC + lax.axis_index("c")
        ins, outs = (in0, in1), (out0, out1)
        isems, osems = (is0, is1), (os0, os1)
        l_vec = lax.iota(jnp.int32, L)
        row_vecs = [l_vec + k * L for k in range(D // L)]

        def vstrip(s_idx):
            return jnp.minimum(wid + s_idx * NW, n_full - 1) * 128

        def fire_in(s_idx, p):
            pltpu.async_copy(wt_hbm.at[:, pl.ds(vstrip(s_idx), 128)],
                             ins[p], isems[p])

        def wait_in(s_idx, p):
            pltpu.make_async_copy(wt_hbm.at[:, pl.ds(vstrip(s_idx), 128)],
                                  ins[p], isems[p]).wait()

        def transpose(p):
            def row_body(j_outer, carry2):
                for jj in range(16):
                    j = j_outer * 16 + jj
                    col_vec = (j + l_vec) & 127
                    sbase = (col_vec << 6) + l_vec
                    for k in range(D // L):
                        vals = plsc.load_gather(ins[p], [row_vecs[k], col_vec])
                        plsc.store_scatter(outs[p], [sbase + k * L], vals)
                return carry2

            lax.fori_loop(0, 8, row_body, 0)

        def fire_out(s_idx, p):
            pltpu.async_copy(outs[p], lin_hbm.at[pl.ds(vstrip(s_idx) * D,
                                                       128 * D)], osems[p])

        def wait_out(p):
            pltpu.make_async_copy(outs[p], lin_hbm.at[pl.ds(0, 128 * D)],
                                  osems[p]).wait()

        fire_in(0, 0)
        for p in range(2):
            wait_in(p, p)
            fire_in(p + 1, 1 - p)
            transpose(p)
            fire_out(p, p)

        def body(t2, carry):
            for p in range(2):
                s_idx = 2 * t2 + p
                wait_in(s_idx, p)
                fire_in(s_idx + 1, 1 - p)
                wait_out(p)
                transpose(p)
                fire_out(s_idx, p)
            return carry

        lax.fori_loop(1, T2, body, 0)
        wait_in(2 * T2, 0)
        wait_out(0)
        wait_out(1)

        if n_tail:
            @pl.when(wid == 0)
            def _():
                pltpu.sync_copy(tail_hbm, out0.at[pl.ds(0, n_tail * D)])
                pltpu.sync_copy(out0.at[pl.ds(0, n_tail * D)],
                                lin_hbm.at[pl.ds(n_full * 128 * D, n_tail * D)])

    return format_kernel


@functools.lru_cache(maxsize=None)
def _build_gather(B, D, NC, NS):
    NW = NC * NS
    b_per_w = B // NW
    S = b_per_w // R
    ngroups = S // G
    T = ngroups // NBANKS

    mesh = plsc.VectorSubcoreMesh(core_axis_name="c", subcore_axis_name="s")

    @functools.partial(
        pl.kernel,
        mesh=mesh,
        out_type=jax.ShapeDtypeStruct((B, D), jnp.float32),
        scratch_types=(
            [pltpu.VMEM((S * R,), jnp.int32)]
            + [pltpu.VMEM((G * R, D), jnp.float32)] * NBANKS
            + [pltpu.SemaphoreType.DMA] * (2 * NBANKS)
        ),
        compiler_params=pltpu.CompilerParams(use_tc_tiling_on_sc=False),
    )
    def gather_kernel(table_hbm, idx_hbm, out_hbm, idx_v, *bufs):
        rows = bufs[:NBANKS]
        gsems = bufs[NBANKS:2 * NBANKS]
        osems = bufs[2 * NBANKS:]
        wid = lax.axis_index("s") * NC + lax.axis_index("c")
        base = wid * b_per_w
        pltpu.sync_copy(idx_hbm.at[pl.ds(base, b_per_w)], idx_v)

        def fire_gathers(g, k):
            for b in range(G):
                s = jnp.minimum(g * G + b, S - 1)
                pltpu.async_copy(table_hbm.at[idx_v.at[pl.ds(s * R, R)]],
                                 rows[k].at[pl.ds(b * R, R)], gsems[k])

        def wait_gathers(g, k):
            for b in range(G):
                s = jnp.minimum(g * G + b, S - 1)
                pltpu.make_async_copy(table_hbm.at[idx_v.at[pl.ds(s * R, R)]],
                                      rows[k].at[pl.ds(b * R, R)], gsems[k]).wait()

        def fire_outs(g, k):
            pltpu.async_copy(rows[k], out_hbm.at[pl.ds(base + g * G * R, G * R)],
                             osems[k])

        def wait_outs(g, k):
            pltpu.make_async_copy(rows[k], out_hbm.at[pl.ds(base + g * G * R, G * R)],
                                  osems[k]).wait()

        for k in range(NBANKS):
            fire_gathers(k, k)

        def body(t, carry):
            g0 = NBANKS * t
            for k in range(NBANKS):
                wait_gathers(g0 + k, k)
                fire_outs(g0 + k, k)
            for k in range(NBANKS):
                wait_outs(g0 + k, k)
                fire_gathers(g0 + NBANKS + k, k)
            return carry

        lax.fori_loop(0, T, body, 0)
        for k in range(NBANKS):
            wait_gathers(NBANKS * T + k, k)

    return gather_kernel


def kernel(token_ids, weight):
    B = token_ids.shape[0] * token_ids.shape[1]
    V, D = weight.shape
    info = plsc.get_sparse_core_info()
    NC, NS = info.num_cores, info.num_subcores
    idx = token_ids.reshape(-1).astype(jnp.int32)
    n_full = V // 128
    tail = weight[n_full * 128:, :].reshape(-1)
    w_lin = _build_format(V, D, NC, NS)(weight.T, tail)
    out = _build_gather(B, D, NC, NS)(w_lin.reshape(V, D), idx)
    return out.reshape(token_ids.shape[0], token_ids.shape[1], D)
